# baseline (speedup 1.0000x reference)
import numpy as np
import jax
import jax.numpy as jnp
from jax import lax
from jax.experimental import pallas as pl
from jax.experimental.pallas import tpu as pltpu

N = 4096
B1 = 256
B2 = 512
NB2 = N // B2
INT_BIG = 2**31 - 1


def _pair_sequence():
    seq = []
    for i in range(NB2):
        for j in range(i, NB2):
            seq.append((i, j, i, j, 0))
            if j > i:
                seq.append((j, i, i, j, 1))
    return np.asarray(seq, dtype=np.int32)


_SEQ = _pair_sequence()


def _pass1_body(adj_ref, mg_ref, adjnew_ref, deg_ref, pmin_ref):
    i = pl.program_id(0)
    a = adj_ref[...]
    m = mg_ref[...]
    adjnew_ref[...] = a
    deg_ref[0, pl.ds(i * B1, B1)] = jnp.sum(a, axis=1)
    tmin = jnp.min(m * (1.0 - 2.0 * a))

    @pl.when(i == 0)
    def _():
        pmin_ref[0, 0] = tmin

    @pl.when(i > 0)
    def _():
        pmin_ref[0, 0] = jnp.minimum(pmin_ref[0, 0], tmin)


def _pass2_body(seq_ref, adj_ref, acr_ref, acc_ref, mg_ref, deg_ref, pmin_ref,
                am_ref, ms_ref, bestv_ref, besti_ref):
    t = pl.program_id(0)
    rblk = seq_ref[t, 0]
    cblk = seq_ref[t, 1]
    mirror = seq_ref[t, 4] == 1

    a0 = adj_ref[...]
    acr = acr_ref[...]
    acc = acc_ref[...]
    mg = mg_ref[...]

    a = jnp.where(mirror, jnp.transpose(a0), a0)
    acs0 = acr + jnp.transpose(acc)
    acs = jnp.where(mirror, jnp.transpose(acs0), acs0)

    rows = lax.broadcasted_iota(jnp.int32, (B2, B2), 0) + rblk * B2
    cols = lax.broadcasted_iota(jnp.int32, (B2, B2), 1) + cblk * B2
    acs = jnp.where(rows == cols, 0.0, acs)
    acs = jnp.clip(acs, -1.0, 1.0)
    am_ref[...] = jnp.clip(a + acs, 0.0, 1.0)

    d1r = (deg_ref[0, pl.ds(rblk * B2, B2)] == 1.0).astype(jnp.float32)
    d1c = (deg_ref[0, pl.ds(cblk * B2, B2)] == 1.0).astype(jnp.float32)
    maskv = a * (d1r[:, None] + d1c[None, :])
    s2 = mg * (1.0 - 2.0 * a) - pmin_ref[0, 0]
    ms = s2 * maskv
    ms_ref[...] = ms

    tmax = jnp.max(ms)
    cand = jnp.min(jnp.where(ms == tmax, rows * N + cols, INT_BIG))

    @pl.when(t == 0)
    def _():
        bestv_ref[0, 0] = -1.0
        besti_ref[0, 0] = INT_BIG

    @pl.when((tmax > bestv_ref[0, 0])
             | ((tmax == bestv_ref[0, 0]) & (cand < besti_ref[0, 0])))
    def _():
        bestv_ref[0, 0] = tmax
        besti_ref[0, 0] = cand


def _flip_body(pos_ref, nv_ref, adjin_ref, out_ref):
    k = pl.program_id(0)
    r0 = (pos_ref[k, 0] // 8) * 8
    c0 = (pos_ref[k, 1] // 128) * 128
    r = pos_ref[0, 0]
    c = pos_ref[0, 1]
    rows = lax.broadcasted_iota(jnp.int32, (8, 128), 0) + r0
    cols = lax.broadcasted_iota(jnp.int32, (8, 128), 1) + c0
    hit = ((rows == r) & (cols == c)) | ((rows == c) & (cols == r))
    out_ref[...] = jnp.where(hit, nv_ref[0, 0], adjin_ref[...])


def kernel(adj, adj_changes, meta_grad, feature_matrix, labels, train_ids, val_ids):
    del feature_matrix, labels, train_ids, val_ids

    adj_new0, deg, pmin = pl.pallas_call(
        _pass1_body,
        grid=(N // B1,),
        in_specs=[
            pl.BlockSpec((B1, N), lambda i: (i, 0)),
            pl.BlockSpec((B1, N), lambda i: (i, 0)),
        ],
        out_specs=[
            pl.BlockSpec((B1, N), lambda i: (i, 0)),
            pl.BlockSpec((1, N), lambda i: (0, 0)),
            pl.BlockSpec(memory_space=pltpu.SMEM),
        ],
        out_shape=[
            jax.ShapeDtypeStruct((N, N), jnp.float32),
            jax.ShapeDtypeStruct((1, N), jnp.float32),
            jax.ShapeDtypeStruct((1, 1), jnp.float32),
        ],
    )(adj, meta_grad)

    seq = jnp.asarray(_SEQ)
    adj_modified, masked_scores, bestv, besti = pl.pallas_call(
        _pass2_body,
        grid_spec=pltpu.PrefetchScalarGridSpec(
            num_scalar_prefetch=1,
            grid=(_SEQ.shape[0],),
            in_specs=[
                pl.BlockSpec((B2, B2), lambda t, s: (s[t, 2], s[t, 3])),
                pl.BlockSpec((B2, B2), lambda t, s: (s[t, 2], s[t, 3])),
                pl.BlockSpec((B2, B2), lambda t, s: (s[t, 3], s[t, 2])),
                pl.BlockSpec((B2, B2), lambda t, s: (s[t, 0], s[t, 1])),
                pl.BlockSpec((1, N), lambda t, s: (0, 0)),
                pl.BlockSpec(memory_space=pltpu.SMEM),
            ],
            out_specs=[
                pl.BlockSpec((B2, B2), lambda t, s: (s[t, 0], s[t, 1])),
                pl.BlockSpec((B2, B2), lambda t, s: (s[t, 0], s[t, 1])),
                pl.BlockSpec(memory_space=pltpu.SMEM),
                pl.BlockSpec(memory_space=pltpu.SMEM),
            ],
        ),
        out_shape=[
            jax.ShapeDtypeStruct((N, N), jnp.float32),
            jax.ShapeDtypeStruct((N, N), jnp.float32),
            jax.ShapeDtypeStruct((1, 1), jnp.float32),
            jax.ShapeDtypeStruct((1, 1), jnp.int32),
        ],
    )(seq, adj, adj_changes, adj_changes, meta_grad, deg, pmin)

    flat = besti[0, 0]
    r = flat // N
    c = flat % N
    pos = jnp.stack([jnp.stack([r, c]), jnp.stack([c, r])]).astype(jnp.int32)
    new_val = jnp.where(bestv[0, 0] > 0.0, 0.0, 1.0).reshape(1, 1).astype(jnp.float32)

    adj_new = pl.pallas_call(
        _flip_body,
        grid_spec=pltpu.PrefetchScalarGridSpec(
            num_scalar_prefetch=1,
            grid=(2,),
            in_specs=[
                pl.BlockSpec(memory_space=pltpu.SMEM),
                pl.BlockSpec((8, 128), lambda k, pos_ref: (pos_ref[k, 0] // 8, pos_ref[k, 1] // 128)),
            ],
            out_specs=pl.BlockSpec((8, 128), lambda k, pos_ref: (pos_ref[k, 0] // 8, pos_ref[k, 1] // 128)),
        ),
        out_shape=jax.ShapeDtypeStruct((N, N), jnp.float32),
        input_output_aliases={2: 0},
    )(pos, new_val, adj_new0)

    return adj_new, adj_modified, masked_scores


# speedup vs baseline: 1.0536x; 1.0536x over previous
import numpy as np
import jax
import jax.numpy as jnp
from jax import lax
from jax.experimental import pallas as pl
from jax.experimental.pallas import tpu as pltpu

N = 4096
B1 = 256
B2 = 512
NB2 = N // B2
INT_BIG = 2**31 - 1


def _pair_sequence():
    seq = []
    for i in range(NB2):
        for j in range(i, NB2):
            seq.append((i, j, i, j, 0))
            if j > i:
                seq.append((j, i, i, j, 1))
    return np.asarray(seq, dtype=np.int32)


_SEQ = _pair_sequence()


def _pass1_body(adj_ref, mg_ref, adjnew_ref, deg_ref, pmin_ref):
    i = pl.program_id(0)
    a = adj_ref[...]
    m = mg_ref[...]
    adjnew_ref[...] = a
    deg_ref[0, pl.ds(i * B1, B1)] = jnp.sum(a, axis=1)
    tmin = jnp.min(m * (1.0 - 2.0 * a))

    @pl.when(i == 0)
    def _():
        pmin_ref[0, 0] = tmin

    @pl.when(i > 0)
    def _():
        pmin_ref[0, 0] = jnp.minimum(pmin_ref[0, 0], tmin)


def _pass2_body(seq_ref, adj_ref, acr_ref, acc_ref, mg_ref, deg_ref, pmin_ref,
                am_ref, ms_ref, bestv_ref, besti_ref):
    t = pl.program_id(0)
    rblk = seq_ref[t, 0]
    cblk = seq_ref[t, 1]
    mirror = seq_ref[t, 4] == 1

    mg = mg_ref[...]

    rows = lax.broadcasted_iota(jnp.int32, (B2, B2), 0) + rblk * B2
    cols = lax.broadcasted_iota(jnp.int32, (B2, B2), 1) + cblk * B2
    d1r = (deg_ref[0, pl.ds(rblk * B2, B2)] == 1.0).astype(jnp.float32)
    d1c = (deg_ref[0, pl.ds(cblk * B2, B2)] == 1.0).astype(jnp.float32)

    def emit(a, acs):
        acs = jnp.where(rows == cols, 0.0, acs)
        acs = jnp.clip(acs, -1.0, 1.0)
        am_ref[...] = jnp.clip(a + acs, 0.0, 1.0)
        maskv = a * (d1r[:, None] + d1c[None, :])
        s2 = mg * (1.0 - 2.0 * a) - pmin_ref[0, 0]
        ms = s2 * maskv
        ms_ref[...] = ms
        return ms

    @pl.when(~mirror)
    def _():
        emit(adj_ref[...], acr_ref[...] + jnp.transpose(acc_ref[...]))

    @pl.when(mirror)
    def _():
        emit(jnp.transpose(adj_ref[...]), acc_ref[...] + jnp.transpose(acr_ref[...]))

    ms = ms_ref[...]
    tmax = jnp.max(ms)
    cand = jnp.min(jnp.where(ms == tmax, rows * N + cols, INT_BIG))

    @pl.when(t == 0)
    def _():
        bestv_ref[0, 0] = -1.0
        besti_ref[0, 0] = INT_BIG

    @pl.when((tmax > bestv_ref[0, 0])
             | ((tmax == bestv_ref[0, 0]) & (cand < besti_ref[0, 0])))
    def _():
        bestv_ref[0, 0] = tmax
        besti_ref[0, 0] = cand


def _flip_body(pos_ref, nv_ref, adjin_ref, out_ref):
    k = pl.program_id(0)
    r0 = (pos_ref[k, 0] // 8) * 8
    c0 = (pos_ref[k, 1] // 128) * 128
    r = pos_ref[0, 0]
    c = pos_ref[0, 1]
    rows = lax.broadcasted_iota(jnp.int32, (8, 128), 0) + r0
    cols = lax.broadcasted_iota(jnp.int32, (8, 128), 1) + c0
    hit = ((rows == r) & (cols == c)) | ((rows == c) & (cols == r))
    out_ref[...] = jnp.where(hit, nv_ref[0, 0], adjin_ref[...])


def kernel(adj, adj_changes, meta_grad, feature_matrix, labels, train_ids, val_ids):
    del feature_matrix, labels, train_ids, val_ids

    adj_new0, deg, pmin = pl.pallas_call(
        _pass1_body,
        grid=(N // B1,),
        in_specs=[
            pl.BlockSpec((B1, N), lambda i: (i, 0)),
            pl.BlockSpec((B1, N), lambda i: (i, 0)),
        ],
        out_specs=[
            pl.BlockSpec((B1, N), lambda i: (i, 0)),
            pl.BlockSpec((1, N), lambda i: (0, 0)),
            pl.BlockSpec(memory_space=pltpu.SMEM),
        ],
        out_shape=[
            jax.ShapeDtypeStruct((N, N), jnp.float32),
            jax.ShapeDtypeStruct((1, N), jnp.float32),
            jax.ShapeDtypeStruct((1, 1), jnp.float32),
        ],
    )(adj, meta_grad)

    seq = jnp.asarray(_SEQ)
    adj_modified, masked_scores, bestv, besti = pl.pallas_call(
        _pass2_body,
        grid_spec=pltpu.PrefetchScalarGridSpec(
            num_scalar_prefetch=1,
            grid=(_SEQ.shape[0],),
            in_specs=[
                pl.BlockSpec((B2, B2), lambda t, s: (s[t, 2], s[t, 3])),
                pl.BlockSpec((B2, B2), lambda t, s: (s[t, 2], s[t, 3])),
                pl.BlockSpec((B2, B2), lambda t, s: (s[t, 3], s[t, 2])),
                pl.BlockSpec((B2, B2), lambda t, s: (s[t, 0], s[t, 1])),
                pl.BlockSpec((1, N), lambda t, s: (0, 0)),
                pl.BlockSpec(memory_space=pltpu.SMEM),
            ],
            out_specs=[
                pl.BlockSpec((B2, B2), lambda t, s: (s[t, 0], s[t, 1])),
                pl.BlockSpec((B2, B2), lambda t, s: (s[t, 0], s[t, 1])),
                pl.BlockSpec(memory_space=pltpu.SMEM),
                pl.BlockSpec(memory_space=pltpu.SMEM),
            ],
        ),
        out_shape=[
            jax.ShapeDtypeStruct((N, N), jnp.float32),
            jax.ShapeDtypeStruct((N, N), jnp.float32),
            jax.ShapeDtypeStruct((1, 1), jnp.float32),
            jax.ShapeDtypeStruct((1, 1), jnp.int32),
        ],
    )(seq, adj, adj_changes, adj_changes, meta_grad, deg, pmin)

    flat = besti[0, 0]
    r = flat // N
    c = flat % N
    pos = jnp.stack([jnp.stack([r, c]), jnp.stack([c, r])]).astype(jnp.int32)
    new_val = jnp.where(bestv[0, 0] > 0.0, 0.0, 1.0).reshape(1, 1).astype(jnp.float32)

    adj_new = pl.pallas_call(
        _flip_body,
        grid_spec=pltpu.PrefetchScalarGridSpec(
            num_scalar_prefetch=1,
            grid=(2,),
            in_specs=[
                pl.BlockSpec(memory_space=pltpu.SMEM),
                pl.BlockSpec((8, 128), lambda k, pos_ref: (pos_ref[k, 0] // 8, pos_ref[k, 1] // 128)),
            ],
            out_specs=pl.BlockSpec((8, 128), lambda k, pos_ref: (pos_ref[k, 0] // 8, pos_ref[k, 1] // 128)),
        ),
        out_shape=jax.ShapeDtypeStruct((N, N), jnp.float32),
        input_output_aliases={2: 0},
    )(pos, new_val, adj_new0)

    return adj_new, adj_modified, masked_scores


# speedup vs baseline: 1.1775x; 1.1176x over previous
import jax
import jax.numpy as jnp
from jax import lax
from jax.experimental import pallas as pl
from jax.experimental.pallas import tpu as pltpu

N = 4096
B1 = 512
B2 = 256
INT_BIG = 2**31 - 1


def _pass1_body(adj_ref, mg_ref, adjnew_ref, deg_ref, pmin_ref):
    i = pl.program_id(0)
    a = adj_ref[...]
    m = mg_ref[...]
    adjnew_ref[...] = a
    deg_ref[0, pl.ds(i * B1, B1)] = jnp.sum(a, axis=1)
    tmin = jnp.min(m * (1.0 - 2.0 * a))

    @pl.when(i == 0)
    def _():
        pmin_ref[0, 0] = tmin

    @pl.when(i > 0)
    def _():
        pmin_ref[0, 0] = jnp.minimum(pmin_ref[0, 0], tmin)


def _pass2_body(adj_ref, acr_ref, acc_ref, mg_ref, deg_ref, pmin_ref,
                am_ref, ms_ref, bestv_ref, besti_ref):
    i = pl.program_id(0)
    a = adj_ref[...]
    acr = acr_ref[...]
    acc = acc_ref[...]
    mg = mg_ref[...]

    rows = lax.broadcasted_iota(jnp.int32, (B2, N), 0) + i * B2
    cols = lax.broadcasted_iota(jnp.int32, (B2, N), 1)

    acs = acr + jnp.transpose(acc)
    acs = jnp.where(rows == cols, 0.0, acs)
    acs = jnp.clip(acs, -1.0, 1.0)
    am_ref[...] = jnp.clip(a + acs, 0.0, 1.0)

    deg = deg_ref[0, :]
    d1c = (deg == 1.0).astype(jnp.float32)
    d1r = (deg_ref[0, pl.ds(i * B2, B2)] == 1.0).astype(jnp.float32)
    maskv = a * (d1r[:, None] + d1c[None, :])
    s2 = mg * (1.0 - 2.0 * a) - pmin_ref[0, 0]
    ms = s2 * maskv
    ms_ref[...] = ms

    tmax = jnp.max(ms)
    cand = jnp.min(jnp.where(ms == tmax, rows * N + cols, INT_BIG))

    @pl.when(i == 0)
    def _():
        bestv_ref[0, 0] = -1.0
        besti_ref[0, 0] = 0

    @pl.when(tmax > bestv_ref[0, 0])
    def _():
        bestv_ref[0, 0] = tmax
        besti_ref[0, 0] = cand


def _flip_body(pos_ref, nv_ref, adjin_ref, out_ref):
    k = pl.program_id(0)
    r0 = (pos_ref[k, 0] // 8) * 8
    c0 = (pos_ref[k, 1] // 128) * 128
    r = pos_ref[0, 0]
    c = pos_ref[0, 1]
    rows = lax.broadcasted_iota(jnp.int32, (8, 128), 0) + r0
    cols = lax.broadcasted_iota(jnp.int32, (8, 128), 1) + c0
    hit = ((rows == r) & (cols == c)) | ((rows == c) & (cols == r))
    out_ref[...] = jnp.where(hit, nv_ref[0, 0], adjin_ref[...])


def kernel(adj, adj_changes, meta_grad, feature_matrix, labels, train_ids, val_ids):
    del feature_matrix, labels, train_ids, val_ids

    adj_new0, deg, pmin = pl.pallas_call(
        _pass1_body,
        grid=(N // B1,),
        in_specs=[
            pl.BlockSpec((B1, N), lambda i: (i, 0)),
            pl.BlockSpec((B1, N), lambda i: (i, 0)),
        ],
        out_specs=[
            pl.BlockSpec((B1, N), lambda i: (i, 0)),
            pl.BlockSpec((1, N), lambda i: (0, 0)),
            pl.BlockSpec(memory_space=pltpu.SMEM),
        ],
        out_shape=[
            jax.ShapeDtypeStruct((N, N), jnp.float32),
            jax.ShapeDtypeStruct((1, N), jnp.float32),
            jax.ShapeDtypeStruct((1, 1), jnp.float32),
        ],
    )(adj, meta_grad)

    adj_modified, masked_scores, bestv, besti = pl.pallas_call(
        _pass2_body,
        grid=(N // B2,),
        in_specs=[
            pl.BlockSpec((B2, N), lambda i: (i, 0)),
            pl.BlockSpec((B2, N), lambda i: (i, 0)),
            pl.BlockSpec((N, B2), lambda i: (0, i)),
            pl.BlockSpec((B2, N), lambda i: (i, 0)),
            pl.BlockSpec((1, N), lambda i: (0, 0)),
            pl.BlockSpec(memory_space=pltpu.SMEM),
        ],
        out_specs=[
            pl.BlockSpec((B2, N), lambda i: (i, 0)),
            pl.BlockSpec((B2, N), lambda i: (i, 0)),
            pl.BlockSpec(memory_space=pltpu.SMEM),
            pl.BlockSpec(memory_space=pltpu.SMEM),
        ],
        out_shape=[
            jax.ShapeDtypeStruct((N, N), jnp.float32),
            jax.ShapeDtypeStruct((N, N), jnp.float32),
            jax.ShapeDtypeStruct((1, 1), jnp.float32),
            jax.ShapeDtypeStruct((1, 1), jnp.int32),
        ],
    )(adj, adj_changes, adj_changes, meta_grad, deg, pmin)

    flat = besti[0, 0]
    r = flat // N
    c = flat % N
    pos = jnp.stack([jnp.stack([r, c]), jnp.stack([c, r])]).astype(jnp.int32)
    new_val = jnp.where(bestv[0, 0] > 0.0, 0.0, 1.0).reshape(1, 1).astype(jnp.float32)

    adj_new = pl.pallas_call(
        _flip_body,
        grid_spec=pltpu.PrefetchScalarGridSpec(
            num_scalar_prefetch=1,
            grid=(2,),
            in_specs=[
                pl.BlockSpec(memory_space=pltpu.SMEM),
                pl.BlockSpec((8, 128), lambda k, pos_ref: (pos_ref[k, 0] // 8, pos_ref[k, 1] // 128)),
            ],
            out_specs=pl.BlockSpec((8, 128), lambda k, pos_ref: (pos_ref[k, 0] // 8, pos_ref[k, 1] // 128)),
        ),
        out_shape=jax.ShapeDtypeStruct((N, N), jnp.float32),
        input_output_aliases={2: 0},
    )(pos, new_val, adj_new0)

    return adj_new, adj_modified, masked_scores
